# Initial kernel scaffold; baseline (speedup 1.0000x reference)
import jax
import jax.numpy as jnp
from jax.experimental import pallas as pl


def kernel(x, edge_index, edge_attr, batch, W1, b1, W2, b2, W3, b3, Wfc, bfc):
    raise NotImplementedError("write your pallas kernel here")


# speedup vs baseline: 6.1238x; 6.1238x over previous
import dataclasses
import functools

import jax
import jax.numpy as jnp
from jax import lax
from jax.experimental import pallas as pl
from jax.experimental.pallas import tpu as pltpu
from jax.experimental.pallas import tpu_sc as plsc

N = 10000
E = 320000
D = 128
NCLS = 16
NC, NS = 2, 16
NW = NC * NS
CHUNK = 128
EPAD = 327680
NCH = EPAD // (NW * CHUNK)
NPAD = 10240
RPS = NPAD // NS

_mesh = plsc.VectorSubcoreMesh(core_axis_name="c", subcore_axis_name="s")

_sc_params = pltpu.CompilerParams()
if "needs_layout_passes" in pltpu.CompilerParams.__dataclass_fields__:
    _sc_params = dataclasses.replace(_sc_params, needs_layout_passes=False)


def _full16(v):
    return jnp.full((16,), v, jnp.int32)


@functools.partial(
    pl.kernel,
    out_type=jax.ShapeDtypeStruct((NC, NPAD, 16), jnp.float32),
    mesh=_mesh,
    scratch_types=[
        pltpu.VMEM((NCH, CHUNK), jnp.int32),
        pltpu.VMEM((NCH, CHUNK), jnp.float32),
        pltpu.VMEM((CHUNK, 16), jnp.float32),
        pltpu.VMEM_SHARED((NPAD, 16), jnp.float32),
    ],
    compiler_params=_sc_params,
)
def _deg_kernel(col_hbm, ew_hbm, zeros_hbm, out_hbm, col_v, ew_v, pay_v, acc_s):
    cid = lax.axis_index("c")
    sid = lax.axis_index("s")
    wid = sid * NC + cid
    pltpu.sync_copy(zeros_hbm.at[pl.ds(sid * RPS, RPS)],
                    acc_s.at[pl.ds(sid * RPS, RPS)])
    pltpu.sync_copy(col_hbm.at[wid], col_v)
    pltpu.sync_copy(ew_hbm.at[wid], ew_v)
    plsc.subcore_barrier()

    @pl.loop(0, NCH)
    def _chunk(j):
        @pl.loop(0, CHUNK)
        def _row(i):
            s = plsc.load_gather(ew_v, [_full16(j), _full16(i)])
            pay_v[i] = s
        pltpu.sync_copy(pay_v, acc_s.at[col_v.at[j]], add=True)

    plsc.subcore_barrier()
    pltpu.sync_copy(acc_s.at[pl.ds(sid * RPS, RPS)],
                    out_hbm.at[cid, pl.ds(sid * RPS, RPS)])


@functools.partial(
    pl.kernel,
    out_type=jax.ShapeDtypeStruct((NC, NPAD, D), jnp.float32),
    mesh=_mesh,
    scratch_types=[
        pltpu.VMEM((NCH, CHUNK), jnp.int32),
        pltpu.VMEM((NCH, CHUNK), jnp.int32),
        pltpu.VMEM((NCH, CHUNK), jnp.float32),
        pltpu.VMEM((CHUNK, D), jnp.float32),
        pltpu.VMEM_SHARED((NPAD, D), jnp.float32),
        pltpu.SemaphoreType.DMA,
    ],
    compiler_params=_sc_params,
)
def _conv_kernel(z_hbm, row_hbm, col_hbm, ew_hbm, zeros_hbm, out_hbm,
                 row_v, col_v, ew_v, g_v, acc_s, sem):
    cid = lax.axis_index("c")
    sid = lax.axis_index("s")
    wid = sid * NC + cid
    pltpu.sync_copy(zeros_hbm.at[pl.ds(sid * RPS, RPS)],
                    acc_s.at[pl.ds(sid * RPS, RPS)])
    pltpu.sync_copy(row_hbm.at[wid], row_v)
    pltpu.sync_copy(col_hbm.at[wid], col_v)
    pltpu.sync_copy(ew_hbm.at[wid], ew_v)
    plsc.subcore_barrier()

    @pl.loop(0, NCH)
    def _chunk(j):
        pltpu.async_copy(z_hbm.at[row_v.at[j]], g_v, sem).wait()

        @pl.loop(0, CHUNK)
        def _row(i):
            s = plsc.load_gather(ew_v, [_full16(j), _full16(i)])
            for k in range(D // 16):
                g_v[i, pl.ds(k * 16, 16)] = g_v[i, pl.ds(k * 16, 16)] * s

        pltpu.sync_copy(g_v, acc_s.at[col_v.at[j]], add=True)

    plsc.subcore_barrier()
    pltpu.sync_copy(acc_s.at[pl.ds(sid * RPS, RPS)],
                    out_hbm.at[cid, pl.ds(sid * RPS, RPS)])


def _dinv(parts_ref):
    deg = parts_ref[0, 0:N, 0:1] + parts_ref[1, 0:N, 0:1] + 1.0
    return lax.rsqrt(deg)


def _prep_body(parts_ref, x_ref, w_ref, z_ref):
    z_ref[...] = _dinv(parts_ref) * jnp.dot(
        x_ref[...], w_ref[...], preferred_element_type=jnp.float32)


def _mid_body(parts_ref, acc_ref, z_ref, b_ref, w_ref, zn_ref):
    dinv = _dinv(parts_ref)
    s = acc_ref[0, 0:N, :] + acc_ref[1, 0:N, :] + z_ref[...]
    h = jnp.maximum(dinv * s + b_ref[...], 0.0)
    zn_ref[...] = dinv * jnp.dot(h, w_ref[...],
                                 preferred_element_type=jnp.float32)


def _final_body(parts_ref, acc_ref, z_ref, b_ref, wfc_ref, bfc_ref, o_ref):
    dinv = _dinv(parts_ref)
    s = acc_ref[0, 0:N, :] + acc_ref[1, 0:N, :] + z_ref[...]
    h = jnp.maximum(dinv * s + b_ref[...], 0.0)
    logits = jnp.dot(h, wfc_ref[...],
                     preferred_element_type=jnp.float32) + bfc_ref[...]
    m = jnp.max(logits, axis=1, keepdims=True)
    e = jnp.exp(logits - m)
    o_ref[...] = e / jnp.sum(e, axis=1, keepdims=True)


_prep_tc = pl.pallas_call(
    _prep_body, out_shape=jax.ShapeDtypeStruct((N, D), jnp.float32))
_mid_tc = pl.pallas_call(
    _mid_body, out_shape=jax.ShapeDtypeStruct((N, D), jnp.float32))
_final_tc = pl.pallas_call(
    _final_body, out_shape=jax.ShapeDtypeStruct((N, NCLS), jnp.float32))


def kernel(x, edge_index, edge_attr, batch, W1, b1, W2, b2, W3, b3, Wfc, bfc):
    del batch
    ei = edge_index.astype(jnp.int32)
    pad = EPAD - E
    row3 = jnp.pad(ei[0], (0, pad)).reshape(NW, NCH, CHUNK)
    col3 = jnp.pad(ei[1], (0, pad)).reshape(NW, NCH, CHUNK)
    ew3 = jnp.pad(edge_attr, (0, pad)).reshape(NW, NCH, CHUNK)
    zeros16 = jnp.zeros((NPAD, 16), jnp.float32)
    zerosd = jnp.zeros((NPAD, D), jnp.float32)
    b1r, b2r, b3r, bfcr = (b.reshape(1, -1) for b in (b1, b2, b3, bfc))

    parts = _deg_kernel(col3, ew3, zeros16)
    z1 = _prep_tc(parts, x, W1)
    acc1 = _conv_kernel(z1, row3, col3, ew3, zerosd)
    z2 = _mid_tc(parts, acc1, z1, b1r, W2)
    acc2 = _conv_kernel(z2, row3, col3, ew3, zerosd)
    z3 = _mid_tc(parts, acc2, z2, b2r, W3)
    acc3 = _conv_kernel(z3, row3, col3, ew3, zerosd)
    return _final_tc(parts, acc3, z3, b3r, Wfc, bfcr)


# speedup vs baseline: 7.9725x; 1.3019x over previous
import dataclasses
import functools

import jax
import jax.numpy as jnp
from jax import lax
from jax.experimental import pallas as pl
from jax.experimental.pallas import tpu as pltpu
from jax.experimental.pallas import tpu_sc as plsc

N = 10000
E = 320000
D = 128
NCLS = 16
NC, NS = 2, 16
NW = NC * NS
CHUNK = 128
EPAD = 327680
NCH = EPAD // (NW * CHUNK)
NPAD = 10240
RPS = NPAD // NS

_mesh = plsc.VectorSubcoreMesh(core_axis_name="c", subcore_axis_name="s")

_sc_params = pltpu.CompilerParams()
if "needs_layout_passes" in pltpu.CompilerParams.__dataclass_fields__:
    _sc_params = dataclasses.replace(_sc_params, needs_layout_passes=False)


def _full16(v):
    return jnp.full((16,), v, jnp.int32)


@functools.partial(
    pl.kernel,
    out_type=jax.ShapeDtypeStruct((NC, NPAD, 16), jnp.float32),
    mesh=_mesh,
    scratch_types=[
        pltpu.VMEM((NCH, CHUNK), jnp.int32),
        pltpu.VMEM((NCH, CHUNK), jnp.float32),
        pltpu.VMEM((CHUNK, 16), jnp.float32),
        pltpu.VMEM_SHARED((NPAD, 16), jnp.float32),
    ],
    compiler_params=_sc_params,
)
def _deg_kernel(col_hbm, ew_hbm, zeros_hbm, out_hbm, col_v, ew_v, pay_v, acc_s):
    cid = lax.axis_index("c")
    sid = lax.axis_index("s")
    wid = sid * NC + cid
    pltpu.sync_copy(zeros_hbm.at[pl.ds(sid * RPS, RPS)],
                    acc_s.at[pl.ds(sid * RPS, RPS)])
    pltpu.sync_copy(col_hbm.at[wid], col_v)
    pltpu.sync_copy(ew_hbm.at[wid], ew_v)
    plsc.subcore_barrier()

    @pl.loop(0, NCH)
    def _chunk(j):
        @pl.loop(0, CHUNK)
        def _row(i):
            s = plsc.load_gather(ew_v, [_full16(j), _full16(i)])
            pay_v[i] = s
        pltpu.sync_copy(pay_v, acc_s.at[col_v.at[j]], add=True)

    plsc.subcore_barrier()
    pltpu.sync_copy(acc_s.at[pl.ds(sid * RPS, RPS)],
                    out_hbm.at[cid, pl.ds(sid * RPS, RPS)])


NBUF = 2
MRING = 5
PERIOD = 10


@functools.partial(
    pl.kernel,
    out_type=jax.ShapeDtypeStruct((NC, NPAD, D), jnp.float32),
    mesh=_mesh,
    scratch_types=[
        pltpu.VMEM((NCH, CHUNK), jnp.int32),
        pltpu.VMEM((MRING, CHUNK), jnp.int32),
        pltpu.VMEM((MRING, CHUNK), jnp.float32),
        pltpu.VMEM((CHUNK, D), jnp.float32),
        pltpu.VMEM((CHUNK, D), jnp.float32),
        pltpu.VMEM_SHARED((NPAD, D), jnp.float32),
    ] + [pltpu.SemaphoreType.DMA] * (NBUF + 2 * MRING),
    compiler_params=_sc_params,
)
def _conv_kernel(z_hbm, row_hbm, ew_hbm, col_hbm, zeros_hbm, out_hbm,
                 col_v, row_v, ew_v, g0_v, g1_v, acc_s, *sems):
    gsem = sems[:NBUF]
    rsem = sems[NBUF:NBUF + MRING]
    wsem = sems[NBUF + MRING:]
    gbufs = (g0_v, g1_v)
    cid = lax.axis_index("c")
    sid = lax.axis_index("s")
    wid = sid * NC + cid
    pltpu.sync_copy(zeros_hbm.at[pl.ds(sid * RPS, RPS)],
                    acc_s.at[pl.ds(sid * RPS, RPS)])
    pltpu.sync_copy(col_hbm.at[wid], col_v)
    for c in range(4):
        pltpu.async_copy(row_hbm.at[wid, c], row_v.at[c], rsem[c])
        pltpu.async_copy(ew_hbm.at[wid, c], ew_v.at[c], wsem[c])
    for c in range(NBUF):
        pltpu.make_async_copy(row_hbm.at[wid, c], row_v.at[c],
                              rsem[c]).wait()
        pltpu.async_copy(z_hbm.at[row_v.at[c]], gbufs[c], gsem[c])
    plsc.subcore_barrier()

    @pl.loop(0, NCH, step=PERIOD)
    def _outer(j):
        for b in range(PERIOD):
            cp = j + b
            gv = gbufs[b % NBUF]
            gs = gsem[b % NBUF]
            mb = b % MRING
            mf = (b + 4) % MRING

            @pl.when(cp + 4 < NCH)
            def _meta():
                pltpu.async_copy(row_hbm.at[wid, cp + 4], row_v.at[mf],
                                 rsem[mf])
                pltpu.async_copy(ew_hbm.at[wid, cp + 4], ew_v.at[mf],
                                 wsem[mf])

            pltpu.make_async_copy(z_hbm.at[row_v.at[mb]], gv, gs).wait()
            pltpu.make_async_copy(ew_hbm.at[wid, cp], ew_v.at[mb],
                                  wsem[mb]).wait()

            @pl.loop(0, CHUNK)
            def _row(i):
                s = plsc.load_gather(ew_v, [_full16(mb), _full16(i)])
                for k in range(D // 16):
                    gv[i, pl.ds(k * 16, 16)] = gv[i, pl.ds(k * 16, 16)] * s

            pltpu.sync_copy(gv, acc_s.at[col_v.at[cp]], add=True)

            @pl.when(cp + 2 < NCH)
            def _refill():
                mn = (b + 2) % MRING
                pltpu.make_async_copy(row_hbm.at[wid, cp + 2],
                                      row_v.at[mn], rsem[mn]).wait()
                pltpu.async_copy(z_hbm.at[row_v.at[mn]], gv, gs)

    plsc.subcore_barrier()
    pltpu.sync_copy(acc_s.at[pl.ds(sid * RPS, RPS)],
                    out_hbm.at[cid, pl.ds(sid * RPS, RPS)])


def _dinv(parts_ref):
    deg = parts_ref[0, 0:N, 0:1] + parts_ref[1, 0:N, 0:1] + 1.0
    return lax.rsqrt(deg)


def _prep_body(parts_ref, x_ref, w_ref, z_ref):
    z_ref[...] = _dinv(parts_ref) * jnp.dot(
        x_ref[...], w_ref[...], preferred_element_type=jnp.float32)


def _mid_body(parts_ref, acc_ref, z_ref, b_ref, w_ref, zn_ref):
    dinv = _dinv(parts_ref)
    s = acc_ref[0, 0:N, :] + acc_ref[1, 0:N, :] + z_ref[...]
    h = jnp.maximum(dinv * s + b_ref[...], 0.0)
    zn_ref[...] = dinv * jnp.dot(h, w_ref[...],
                                 preferred_element_type=jnp.float32)


def _final_body(parts_ref, acc_ref, z_ref, b_ref, wfc_ref, bfc_ref, o_ref):
    dinv = _dinv(parts_ref)
    s = acc_ref[0, 0:N, :] + acc_ref[1, 0:N, :] + z_ref[...]
    h = jnp.maximum(dinv * s + b_ref[...], 0.0)
    logits = jnp.dot(h, wfc_ref[...],
                     preferred_element_type=jnp.float32) + bfc_ref[...]
    m = jnp.max(logits, axis=1, keepdims=True)
    e = jnp.exp(logits - m)
    o_ref[...] = e / jnp.sum(e, axis=1, keepdims=True)


_prep_tc = pl.pallas_call(
    _prep_body, out_shape=jax.ShapeDtypeStruct((N, D), jnp.float32))
_mid_tc = pl.pallas_call(
    _mid_body, out_shape=jax.ShapeDtypeStruct((N, D), jnp.float32))
_final_tc = pl.pallas_call(
    _final_body, out_shape=jax.ShapeDtypeStruct((N, NCLS), jnp.float32))


def kernel(x, edge_index, edge_attr, batch, W1, b1, W2, b2, W3, b3, Wfc, bfc):
    del batch
    ei = edge_index.astype(jnp.int32)
    pad = EPAD - E
    row3 = jnp.pad(ei[0], (0, pad)).reshape(NW, NCH, CHUNK)
    col3 = jnp.pad(ei[1], (0, pad)).reshape(NW, NCH, CHUNK)
    ew3 = jnp.pad(edge_attr, (0, pad)).reshape(NW, NCH, CHUNK)
    zeros16 = jnp.zeros((NPAD, 16), jnp.float32)
    zerosd = jnp.zeros((NPAD, D), jnp.float32)
    b1r, b2r, b3r, bfcr = (b.reshape(1, -1) for b in (b1, b2, b3, bfc))

    parts = _deg_kernel(col3, ew3, zeros16)
    z1 = _prep_tc(parts, x, W1)
    acc1 = _conv_kernel(z1, row3, ew3, col3, zerosd)
    z2 = _mid_tc(parts, acc1, z1, b1r, W2)
    acc2 = _conv_kernel(z2, row3, ew3, col3, zerosd)
    z3 = _mid_tc(parts, acc2, z2, b2r, W3)
    acc3 = _conv_kernel(z3, row3, ew3, col3, zerosd)
    return _final_tc(parts, acc3, z3, b3r, Wfc, bfcr)


# speedup vs baseline: 8.4480x; 1.0596x over previous
import dataclasses
import functools

import jax
import jax.numpy as jnp
from jax import lax
from jax.experimental import pallas as pl
from jax.experimental.pallas import tpu as pltpu
from jax.experimental.pallas import tpu_sc as plsc

N = 10000
E = 320000
D = 128
NCLS = 16
NC, NS = 2, 16
NW = NC * NS
CHUNK = 128
EPAD = 327680
NCH = EPAD // (NW * CHUNK)
NPAD = 10240
RPS = NPAD // NS

_mesh = plsc.VectorSubcoreMesh(core_axis_name="c", subcore_axis_name="s")

_sc_params = pltpu.CompilerParams()
if "needs_layout_passes" in pltpu.CompilerParams.__dataclass_fields__:
    _sc_params = dataclasses.replace(_sc_params, needs_layout_passes=False)


def _full16(v):
    return jnp.full((16,), v, jnp.int32)


@functools.partial(
    pl.kernel,
    out_type=jax.ShapeDtypeStruct((NC, NPAD, 16), jnp.float32),
    mesh=_mesh,
    scratch_types=[
        pltpu.VMEM((NCH, CHUNK), jnp.int32),
        pltpu.VMEM((NCH, CHUNK), jnp.float32),
        pltpu.VMEM((CHUNK, 16), jnp.float32),
        pltpu.VMEM_SHARED((NPAD, 16), jnp.float32),
    ],
    compiler_params=_sc_params,
)
def _deg_kernel(col_hbm, ew_hbm, zeros_hbm, out_hbm, col_v, ew_v, pay_v, acc_s):
    cid = lax.axis_index("c")
    sid = lax.axis_index("s")
    wid = sid * NC + cid
    pltpu.sync_copy(zeros_hbm.at[pl.ds(sid * RPS, RPS)],
                    acc_s.at[pl.ds(sid * RPS, RPS)])
    pltpu.sync_copy(col_hbm.at[wid], col_v)
    pltpu.sync_copy(ew_hbm.at[wid], ew_v)
    plsc.subcore_barrier()

    @pl.loop(0, NCH)
    def _chunk(j):
        @pl.loop(0, CHUNK)
        def _row(i):
            s = plsc.load_gather(ew_v, [_full16(j), _full16(i)])
            pay_v[i] = s
        pltpu.sync_copy(pay_v, acc_s.at[col_v.at[j]], add=True)

    plsc.subcore_barrier()
    pltpu.sync_copy(acc_s.at[pl.ds(sid * RPS, RPS)],
                    out_hbm.at[cid, pl.ds(sid * RPS, RPS)])


NBUF = 2
MRING = 5
PERIOD = 10
NCH0 = 120
NCH1 = 40
NCHT = NS * (NCH0 + NCH1)


@functools.partial(
    pl.kernel,
    out_type=jax.ShapeDtypeStruct((NC, NPAD, D), jnp.float32),
    mesh=_mesh,
    scratch_types=[
        pltpu.VMEM((MRING, CHUNK), jnp.int32),
        pltpu.VMEM((MRING, CHUNK), jnp.int32),
        pltpu.VMEM((MRING, CHUNK), jnp.float32),
        pltpu.VMEM((CHUNK, D), jnp.float32),
        pltpu.VMEM((CHUNK, D), jnp.float32),
        pltpu.VMEM_SHARED((NPAD, D), jnp.float32),
    ] + [pltpu.SemaphoreType.DMA] * (NBUF + 3 * MRING),
    compiler_params=_sc_params,
)
def _conv_kernel(z_hbm, row_hbm, ew_hbm, col_hbm, zeros_hbm, out_hbm,
                 col_v, row_v, ew_v, g0_v, g1_v, acc_s, *sems):
    gsem = sems[:NBUF]
    rsem = sems[NBUF:NBUF + MRING]
    wsem = sems[NBUF + MRING:NBUF + 2 * MRING]
    csem = sems[NBUF + 2 * MRING:]
    gbufs = (g0_v, g1_v)
    cid = lax.axis_index("c")
    sid = lax.axis_index("s")
    start = pl.multiple_of(
        jnp.where(cid == 0, sid * NCH0, NS * NCH0 + sid * NCH1), 8)
    nch = jnp.where(cid == 0, NCH0, NCH1)
    pltpu.sync_copy(zeros_hbm.at[pl.ds(sid * RPS, RPS)],
                    acc_s.at[pl.ds(sid * RPS, RPS)])
    for c in range(4):
        pltpu.async_copy(row_hbm.at[start + c, 0], row_v.at[c], rsem[c])
        pltpu.async_copy(ew_hbm.at[start + c, 0], ew_v.at[c], wsem[c])
        pltpu.async_copy(col_hbm.at[start + c, 0], col_v.at[c], csem[c])
    for c in range(NBUF):
        pltpu.make_async_copy(row_hbm.at[start + c, 0], row_v.at[c],
                              rsem[c]).wait()
        pltpu.async_copy(z_hbm.at[row_v.at[c]], gbufs[c], gsem[c])
    plsc.subcore_barrier()

    @pl.loop(0, NCH0, step=PERIOD)
    def _outer(j):
        for b in range(PERIOD):
            cp = j + b
            gv = gbufs[b % NBUF]
            gs = gsem[b % NBUF]
            mb = b % MRING
            mf = (b + 4) % MRING

            @pl.when(cp < nch)
            def _phase():
                @pl.when(cp + 4 < nch)
                def _meta():
                    pltpu.async_copy(row_hbm.at[start + cp + 4, 0],
                                     row_v.at[mf], rsem[mf])
                    pltpu.async_copy(ew_hbm.at[start + cp + 4, 0],
                                     ew_v.at[mf], wsem[mf])
                    pltpu.async_copy(col_hbm.at[start + cp + 4, 0],
                                     col_v.at[mf], csem[mf])

                pltpu.make_async_copy(z_hbm.at[row_v.at[mb]], gv,
                                      gs).wait()
                pltpu.make_async_copy(ew_hbm.at[start + cp, 0],
                                      ew_v.at[mb], wsem[mb]).wait()
                pltpu.make_async_copy(col_hbm.at[start + cp, 0],
                                      col_v.at[mb], csem[mb]).wait()

                @pl.loop(0, CHUNK)
                def _row(i):
                    s = plsc.load_gather(ew_v, [_full16(mb), _full16(i)])
                    for k in range(D // 16):
                        gv[i, pl.ds(k * 16, 16)] = (
                            gv[i, pl.ds(k * 16, 16)] * s)

                pltpu.sync_copy(gv, acc_s.at[col_v.at[mb]], add=True)

                @pl.when(cp + 2 < nch)
                def _refill():
                    mn = (b + 2) % MRING
                    pltpu.make_async_copy(row_hbm.at[start + cp + 2, 0],
                                          row_v.at[mn], rsem[mn]).wait()
                    pltpu.async_copy(z_hbm.at[row_v.at[mn]], gv, gs)

    plsc.subcore_barrier()
    pltpu.sync_copy(acc_s.at[pl.ds(sid * RPS, RPS)],
                    out_hbm.at[cid, pl.ds(sid * RPS, RPS)])


def _dinv(parts_ref):
    deg = parts_ref[0, 0:N, 0:1] + parts_ref[1, 0:N, 0:1] + 1.0
    return lax.rsqrt(deg)


def _prep_body(parts_ref, x_ref, w_ref, z_ref):
    z_ref[...] = _dinv(parts_ref) * jnp.dot(
        x_ref[...], w_ref[...], preferred_element_type=jnp.float32)


def _mid_body(parts_ref, acc_ref, z_ref, b_ref, w_ref, zn_ref):
    dinv = _dinv(parts_ref)
    s = acc_ref[0, 0:N, :] + acc_ref[1, 0:N, :] + z_ref[...]
    h = jnp.maximum(dinv * s + b_ref[...], 0.0)
    zn_ref[...] = dinv * jnp.dot(h, w_ref[...],
                                 preferred_element_type=jnp.float32)


def _final_body(parts_ref, acc_ref, z_ref, b_ref, wfc_ref, bfc_ref, o_ref):
    dinv = _dinv(parts_ref)
    s = acc_ref[0, 0:N, :] + acc_ref[1, 0:N, :] + z_ref[...]
    h = jnp.maximum(dinv * s + b_ref[...], 0.0)
    logits = jnp.dot(h, wfc_ref[...],
                     preferred_element_type=jnp.float32) + bfc_ref[...]
    m = jnp.max(logits, axis=1, keepdims=True)
    e = jnp.exp(logits - m)
    o_ref[...] = e / jnp.sum(e, axis=1, keepdims=True)


_prep_tc = pl.pallas_call(
    _prep_body, out_shape=jax.ShapeDtypeStruct((N, D), jnp.float32))
_mid_tc = pl.pallas_call(
    _mid_body, out_shape=jax.ShapeDtypeStruct((N, D), jnp.float32))
_final_tc = pl.pallas_call(
    _final_body, out_shape=jax.ShapeDtypeStruct((N, NCLS), jnp.float32))


def kernel(x, edge_index, edge_attr, batch, W1, b1, W2, b2, W3, b3, Wfc, bfc):
    del batch
    ei = edge_index.astype(jnp.int32)
    pad = EPAD - E
    rowp = jnp.pad(ei[0], (0, pad))
    colp = jnp.pad(ei[1], (0, pad))
    ewp = jnp.pad(edge_attr, (0, pad))
    col3 = colp.reshape(NW, NCH, CHUNK)
    ew3 = ewp.reshape(NW, NCH, CHUNK)
    row_f = rowp.reshape(NCHT, 1, CHUNK)
    col_f = colp.reshape(NCHT, 1, CHUNK)
    ew_f = ewp.reshape(NCHT, 1, CHUNK)
    zeros16 = jnp.zeros((NPAD, 16), jnp.float32)
    zerosd = jnp.zeros((NPAD, D), jnp.float32)
    b1r, b2r, b3r, bfcr = (b.reshape(1, -1) for b in (b1, b2, b3, bfc))

    parts = _deg_kernel(col3, ew3, zeros16)
    z1 = _prep_tc(parts, x, W1)
    acc1 = _conv_kernel(z1, row_f, ew_f, col_f, zerosd)
    z2 = _mid_tc(parts, acc1, z1, b1r, W2)
    acc2 = _conv_kernel(z2, row_f, ew_f, col_f, zerosd)
    z3 = _mid_tc(parts, acc2, z2, b2r, W3)
    acc3 = _conv_kernel(z3, row_f, ew_f, col_f, zerosd)
    return _final_tc(parts, acc3, z3, b3r, Wfc, bfcr)


# speedup vs baseline: 8.4484x; 1.0000x over previous
import dataclasses
import functools

import jax
import jax.numpy as jnp
from jax import lax
from jax.experimental import pallas as pl
from jax.experimental.pallas import tpu as pltpu
from jax.experimental.pallas import tpu_sc as plsc

N = 10000
E = 320000
D = 128
NCLS = 16
NC, NS = 2, 16
NW = NC * NS
CHUNK = 128
EPAD = 327680
NCH = EPAD // (NW * CHUNK)
NPAD = 10240
RPS = NPAD // NS

_mesh = plsc.VectorSubcoreMesh(core_axis_name="c", subcore_axis_name="s")

_sc_params = pltpu.CompilerParams()
if "needs_layout_passes" in pltpu.CompilerParams.__dataclass_fields__:
    _sc_params = dataclasses.replace(_sc_params, needs_layout_passes=False)


def _full16(v):
    return jnp.full((16,), v, jnp.int32)


@functools.partial(
    pl.kernel,
    out_type=jax.ShapeDtypeStruct((NC, NPAD, 16), jnp.float32),
    mesh=_mesh,
    scratch_types=[
        pltpu.VMEM((NCH, CHUNK), jnp.int32),
        pltpu.VMEM((NCH, CHUNK), jnp.float32),
        pltpu.VMEM((CHUNK, 16), jnp.float32),
        pltpu.VMEM_SHARED((NPAD, 16), jnp.float32),
    ],
    compiler_params=_sc_params,
)
def _deg_kernel(col_hbm, ew_hbm, zeros_hbm, out_hbm, col_v, ew_v, pay_v, acc_s):
    cid = lax.axis_index("c")
    sid = lax.axis_index("s")
    wid = sid * NC + cid
    pltpu.sync_copy(zeros_hbm.at[pl.ds(sid * RPS, RPS)],
                    acc_s.at[pl.ds(sid * RPS, RPS)])
    pltpu.sync_copy(col_hbm.at[wid], col_v)
    pltpu.sync_copy(ew_hbm.at[wid], ew_v)
    plsc.subcore_barrier()

    @pl.loop(0, NCH)
    def _chunk(j):
        @pl.loop(0, CHUNK)
        def _row(i):
            s = plsc.load_gather(ew_v, [_full16(j), _full16(i)])
            pay_v[i] = s
        pltpu.sync_copy(pay_v, acc_s.at[col_v.at[j]], add=True)

    plsc.subcore_barrier()
    pltpu.sync_copy(acc_s.at[pl.ds(sid * RPS, RPS)],
                    out_hbm.at[cid, pl.ds(sid * RPS, RPS)])


NBUF = 2
MRING = 5
PERIOD = 10
NCH0 = 120
NCH1 = 40
NCHT = NS * (NCH0 + NCH1)


@functools.partial(
    pl.kernel,
    out_type=jax.ShapeDtypeStruct((NC, NPAD, D), jnp.float32),
    mesh=_mesh,
    scratch_types=[
        pltpu.VMEM((MRING, CHUNK), jnp.int32),
        pltpu.VMEM((MRING, CHUNK), jnp.int32),
        pltpu.VMEM((MRING, CHUNK), jnp.float32),
        pltpu.VMEM((CHUNK, D), jnp.float32),
        pltpu.VMEM((CHUNK, D), jnp.float32),
        pltpu.VMEM_SHARED((NPAD, D), jnp.float32),
    ] + [pltpu.SemaphoreType.DMA] * (NBUF + 3 * MRING),
    compiler_params=_sc_params,
)
def _conv_kernel(z_hbm, row_hbm, ew_hbm, col_hbm, zeros_hbm, out_hbm,
                 col_v, row_v, ew_v, g0_v, g1_v, acc_s, *sems):
    gsem = sems[:NBUF]
    rsem = sems[NBUF:NBUF + MRING]
    wsem = sems[NBUF + MRING:NBUF + 2 * MRING]
    csem = sems[NBUF + 2 * MRING:]
    gbufs = (g0_v, g1_v)
    cid = lax.axis_index("c")
    sid = lax.axis_index("s")
    start = pl.multiple_of(
        jnp.where(cid == 0, sid * NCH0, NS * NCH0 + sid * NCH1), 8)
    nch = jnp.where(cid == 0, NCH0, NCH1)
    pltpu.sync_copy(zeros_hbm.at[pl.ds(sid * RPS, RPS)],
                    acc_s.at[pl.ds(sid * RPS, RPS)])
    for c in range(4):
        pltpu.async_copy(row_hbm.at[start + c, 0], row_v.at[c], rsem[c])
        pltpu.async_copy(ew_hbm.at[start + c, 0], ew_v.at[c], wsem[c])
        pltpu.async_copy(col_hbm.at[start + c, 0], col_v.at[c], csem[c])
    for c in range(NBUF):
        pltpu.make_async_copy(row_hbm.at[start + c, 0], row_v.at[c],
                              rsem[c]).wait()
        pltpu.async_copy(z_hbm.at[row_v.at[c]], gbufs[c], gsem[c])
    plsc.subcore_barrier()

    @pl.loop(0, NCH0, step=PERIOD)
    def _outer(j):
        for b in range(PERIOD):
            cp = j + b
            gv = gbufs[b % NBUF]
            gs = gsem[b % NBUF]
            mb = b % MRING
            mf = (b + 4) % MRING

            @pl.when(cp < nch)
            def _phase():
                @pl.when(cp + 4 < nch)
                def _meta():
                    pltpu.async_copy(row_hbm.at[start + cp + 4, 0],
                                     row_v.at[mf], rsem[mf])
                    pltpu.async_copy(ew_hbm.at[start + cp + 4, 0],
                                     ew_v.at[mf], wsem[mf])
                    pltpu.async_copy(col_hbm.at[start + cp + 4, 0],
                                     col_v.at[mf], csem[mf])

                pltpu.make_async_copy(z_hbm.at[row_v.at[mb]], gv,
                                      gs).wait()
                pltpu.make_async_copy(ew_hbm.at[start + cp, 0],
                                      ew_v.at[mb], wsem[mb]).wait()
                pltpu.make_async_copy(col_hbm.at[start + cp, 0],
                                      col_v.at[mb], csem[mb]).wait()

                @pl.loop(0, CHUNK)
                def _row(i):
                    s = plsc.load_gather(ew_v, [_full16(mb), _full16(i)])
                    for k in range(D // 16):
                        gv[i, pl.ds(k * 16, 16)] = (
                            gv[i, pl.ds(k * 16, 16)] * s)

                pltpu.sync_copy(gv, acc_s.at[col_v.at[mb]], add=True)

                @pl.when(cp + 2 < nch)
                def _refill():
                    mn = (b + 2) % MRING
                    pltpu.make_async_copy(row_hbm.at[start + cp + 2, 0],
                                          row_v.at[mn], rsem[mn]).wait()
                    pltpu.async_copy(z_hbm.at[row_v.at[mn]], gv, gs)

    plsc.subcore_barrier()
    pltpu.sync_copy(acc_s.at[pl.ds(sid * RPS, RPS)],
                    out_hbm.at[cid, pl.ds(sid * RPS, RPS)])


def _dinv(parts_ref):
    deg = parts_ref[0, 0:N, 0:1] + parts_ref[1, 0:N, 0:1] + 1.0
    return lax.rsqrt(deg)


def _prep_body(parts_ref, x_ref, w_ref, z_ref):
    z_ref[...] = _dinv(parts_ref) * jnp.dot(
        x_ref[...], w_ref[...], preferred_element_type=jnp.float32)


def _mid_body(parts_ref, acc_ref, z_ref, b_ref, w_ref, zn_ref):
    dinv = _dinv(parts_ref)
    s = acc_ref[0, 0:N, :] + acc_ref[1, 0:N, :] + z_ref[...]
    h = jnp.maximum(dinv * s + b_ref[...], 0.0)
    zn_ref[...] = dinv * jnp.dot(h, w_ref[...],
                                 preferred_element_type=jnp.float32)


def _final_body(parts_ref, acc_ref, z_ref, b_ref, wfc_ref, bfc_ref, o_ref):
    dinv = _dinv(parts_ref)
    s = acc_ref[0, 0:N, :] + acc_ref[1, 0:N, :] + z_ref[...]
    h = jnp.maximum(dinv * s + b_ref[...], 0.0)
    logits = jnp.dot(h, wfc_ref[...],
                     preferred_element_type=jnp.float32) + bfc_ref[...]
    m = jnp.max(logits, axis=1, keepdims=True)
    e = jnp.exp(logits - m)
    o_ref[...] = e / jnp.sum(e, axis=1, keepdims=True)


_zout = jax.ShapeDtypeStruct((N, D), jnp.float32)
_prep_tc = pl.pallas_call(_prep_body, out_shape=_zout)
_mid_tc = pl.pallas_call(_mid_body, out_shape=_zout)
_final_tc = pl.pallas_call(
    _final_body, out_shape=jax.ShapeDtypeStruct((N, NCLS), jnp.float32))


def kernel(x, edge_index, edge_attr, batch, W1, b1, W2, b2, W3, b3, Wfc, bfc):
    del batch
    ei = edge_index.astype(jnp.int32)
    pad = EPAD - E
    rowp = jnp.pad(ei[0], (0, pad))
    colp = jnp.pad(ei[1], (0, pad))
    ewp = jnp.pad(edge_attr, (0, pad))
    col3 = colp.reshape(NW, NCH, CHUNK)
    ew3 = ewp.reshape(NW, NCH, CHUNK)
    row_f = rowp.reshape(NW * NCH, 1, CHUNK)
    col_f = colp.reshape(NW * NCH, 1, CHUNK)
    ew_f = ewp.reshape(NW * NCH, 1, CHUNK)
    zeros16 = jnp.zeros((NPAD, 16), jnp.float32)
    zerosd = jnp.zeros((NPAD, D), jnp.float32)
    b1r, b2r, b3r, bfcr = (b.reshape(1, -1) for b in (b1, b2, b3, bfc))

    parts = _deg_kernel(col3, ew3, zeros16)
    z1 = _prep_tc(parts, x, W1)
    acc1 = _conv_kernel(z1, row_f, ew_f, col_f, zerosd)
    z2 = _mid_tc(parts, acc1, z1, b1r, W2)
    acc2 = _conv_kernel(z2, row_f, ew_f, col_f, zerosd)
    z3 = _mid_tc(parts, acc2, z2, b2r, W3)
    acc3 = _conv_kernel(z3, row_f, ew_f, col_f, zerosd)
    return _final_tc(parts, acc3, z3, b3r, Wfc, bfcr)
